# baseline (speedup 1.0000x reference)
import functools

import jax
import jax.numpy as jnp
from jax import lax
from jax.experimental import pallas as pl
from jax.experimental.pallas import tpu as pltpu
from jax.experimental.pallas import tpu_sc as plsc

N = 4
GL, GH, GW = 8, 16, 16
NCELL = GL * GH * GW
NCH = 12
P = 512 * 512
NWORKERS = 32
WPV = NWORKERS // N
CH = 4096
CPV = P // CH
CPW = CPV // WPV
LANES = 16


def _sc_body(xy_hbm, rgb_hbm, grids_hbm, out_hbm, grid_v, xy_v, rgb_v, out_v):
    cid = lax.axis_index("c")
    sid = lax.axis_index("s")
    wid = sid * 2 + cid
    view = wid // WPV
    slot = wid % WPV

    pltpu.sync_copy(grids_hbm.at[view], grid_v)

    iota = lax.iota(jnp.int32, LANES)
    iota2 = iota * 2
    iota3 = iota * 3
    coffs = [jnp.full((LANES,), c * NCELL, jnp.int32) for c in range(NCH)]

    def pix_body(i, carry):
        b2 = jnp.full((LANES,), i * (2 * LANES), jnp.int32) + iota2
        b3 = jnp.full((LANES,), i * (3 * LANES), jnp.int32) + iota3
        xv = plsc.load_gather(xy_v, [b2]) * float(GW - 1)
        yv = plsc.load_gather(xy_v, [b2 + 1]) * float(GH - 1)
        rv = plsc.load_gather(rgb_v, [b3])
        gv = plsc.load_gather(rgb_v, [b3 + 1])
        bv = plsc.load_gather(rgb_v, [b3 + 2])
        gray = rv * 0.299 + gv * 0.587 + bv * 0.114
        zv = gray * float(GL - 1)

        x0 = xv.astype(jnp.int32)
        y0 = yv.astype(jnp.int32)
        z0 = zv.astype(jnp.int32)
        wx = xv - x0.astype(jnp.float32)
        wy = yv - y0.astype(jnp.float32)
        wz = zv - z0.astype(jnp.float32)
        x0c = jnp.minimum(x0, GW - 1)
        x1c = jnp.minimum(x0 + 1, GW - 1)
        y0c = jnp.minimum(y0, GH - 1)
        y1c = jnp.minimum(y0 + 1, GH - 1)
        z0c = jnp.minimum(z0, GL - 1)
        z1c = jnp.minimum(z0 + 1, GL - 1)

        r0 = z0c * (GH * GW)
        r1 = z1c * (GH * GW)
        c0 = y0c * GW
        c1 = y1c * GW
        zy00 = r0 + c0
        zy01 = r0 + c1
        zy10 = r1 + c0
        zy11 = r1 + c1
        idxs = [zy00 + x0c, zy00 + x1c,
                zy01 + x0c, zy01 + x1c,
                zy10 + x0c, zy10 + x1c,
                zy11 + x0c, zy11 + x1c]

        ux = 1.0 - wx
        uy = 1.0 - wy
        uz = 1.0 - wz
        wzy00 = uz * uy
        wzy01 = uz * wy
        wzy10 = wz * uy
        wzy11 = wz * wy
        ws = [wzy00 * ux, wzy00 * wx,
              wzy01 * ux, wzy01 * wx,
              wzy10 * ux, wzy10 * wx,
              wzy11 * ux, wzy11 * wx]

        mats = []
        for c in range(NCH):
            acc = ws[0] * plsc.load_gather(grid_v, [coffs[c] + idxs[0]])
            for k in range(1, 8):
                acc = acc + ws[k] * plsc.load_gather(grid_v, [coffs[c] + idxs[k]])
            mats.append(acc)

        outr = mats[0] * rv + mats[1] * gv + mats[2] * bv + mats[3]
        outg = mats[4] * rv + mats[5] * gv + mats[6] * bv + mats[7]
        outb = mats[8] * rv + mats[9] * gv + mats[10] * bv + mats[11]
        plsc.store_scatter(out_v, [b3], outr)
        plsc.store_scatter(out_v, [b3 + 1], outg)
        plsc.store_scatter(out_v, [b3 + 2], outb)
        return carry

    def chunk_body(ci, carry):
        gchunk = (view * WPV + slot) * CPW + ci
        pltpu.sync_copy(xy_hbm.at[gchunk], xy_v)
        pltpu.sync_copy(rgb_hbm.at[gchunk], rgb_v)
        lax.fori_loop(0, CH // LANES, pix_body, 0)
        pltpu.sync_copy(out_v, out_hbm.at[gchunk])
        return carry

    lax.fori_loop(0, CPW, chunk_body, 0)


_bilagrid_sc = functools.partial(
    pl.kernel,
    out_type=jax.ShapeDtypeStruct((N * CPV, 3 * CH), jnp.float32),
    mesh=plsc.VectorSubcoreMesh(core_axis_name="c", subcore_axis_name="s"),
    compiler_params=pltpu.CompilerParams(needs_layout_passes=False),
    scratch_types=[
        pltpu.VMEM((NCH * NCELL,), jnp.float32),
        pltpu.VMEM((2 * CH,), jnp.float32),
        pltpu.VMEM((3 * CH,), jnp.float32),
        pltpu.VMEM((3 * CH,), jnp.float32),
    ],
)(_sc_body)


def kernel(grids, grid_xy, rgb):
    xy = grid_xy.reshape(N * CPV, 2 * CH)
    rgbf = rgb.reshape(N * CPV, 3 * CH)
    g = grids.reshape(N, NCH * NCELL)
    out = _bilagrid_sc(xy, rgbf, g)
    return out.reshape(rgb.shape)


# speedup vs baseline: 11.7897x; 11.7897x over previous
import functools

import jax
import jax.numpy as jnp
from jax import lax
from jax.experimental import pallas as pl
from jax.experimental.pallas import tpu as pltpu
from jax.experimental.pallas import tpu_sc as plsc

N = 4
GL, GH, GW = 8, 16, 16
NCELL = GL * GH * GW
NCH = 12
P = 512 * 512
NWORKERS = 32
WPV = NWORKERS // N
CH = 2048
CPV = P // CH
CPW = CPV // WPV
LANES = 16


def _sc_body(data_hbm, grids_hbm, out_hbm, grid_v, in_v, out_v):
    cid = lax.axis_index("c")
    sid = lax.axis_index("s")
    wid = sid * 2 + cid
    view = wid // WPV
    slot = wid % WPV

    pltpu.sync_copy(grids_hbm.at[view], grid_v)

    coffs = [jnp.full((LANES,), c * NCELL, jnp.int32) for c in range(NCH)]

    def pix_body(i, carry):
        s = pl.ds(i * LANES, LANES)
        xv = in_v[0, s] * float(GW - 1)
        yv = in_v[1, s] * float(GH - 1)
        rv = in_v[2, s]
        gv = in_v[3, s]
        bv = in_v[4, s]
        gray = rv * 0.299 + gv * 0.587 + bv * 0.114
        zv = gray * float(GL - 1)

        x0 = xv.astype(jnp.int32)
        y0 = yv.astype(jnp.int32)
        z0 = zv.astype(jnp.int32)
        wx = xv - x0.astype(jnp.float32)
        wy = yv - y0.astype(jnp.float32)
        wz = zv - z0.astype(jnp.float32)
        x0c = jnp.minimum(x0, GW - 1)
        x1c = jnp.minimum(x0 + 1, GW - 1)
        y0c = jnp.minimum(y0, GH - 1)
        y1c = jnp.minimum(y0 + 1, GH - 1)
        z0c = jnp.minimum(z0, GL - 1)
        z1c = jnp.minimum(z0 + 1, GL - 1)

        r0 = z0c * (GH * GW)
        r1 = z1c * (GH * GW)
        c0 = y0c * GW
        c1 = y1c * GW
        zy00 = r0 + c0
        zy01 = r0 + c1
        zy10 = r1 + c0
        zy11 = r1 + c1
        idxs = [zy00 + x0c, zy00 + x1c,
                zy01 + x0c, zy01 + x1c,
                zy10 + x0c, zy10 + x1c,
                zy11 + x0c, zy11 + x1c]

        ux = 1.0 - wx
        uy = 1.0 - wy
        uz = 1.0 - wz
        wzy00 = uz * uy
        wzy01 = uz * wy
        wzy10 = wz * uy
        wzy11 = wz * wy
        ws = [wzy00 * ux, wzy00 * wx,
              wzy01 * ux, wzy01 * wx,
              wzy10 * ux, wzy10 * wx,
              wzy11 * ux, wzy11 * wx]

        mats = []
        for c in range(NCH):
            acc = ws[0] * plsc.load_gather(grid_v, [coffs[c] + idxs[0]])
            for k in range(1, 8):
                acc = acc + ws[k] * plsc.load_gather(grid_v, [coffs[c] + idxs[k]])
            mats.append(acc)

        out_v[0, s] = mats[0] * rv + mats[1] * gv + mats[2] * bv + mats[3]
        out_v[1, s] = mats[4] * rv + mats[5] * gv + mats[6] * bv + mats[7]
        out_v[2, s] = mats[8] * rv + mats[9] * gv + mats[10] * bv + mats[11]
        return carry

    def chunk_body(ci, carry):
        gchunk = view * CPV + slot * CPW + ci
        pltpu.sync_copy(data_hbm.at[gchunk], in_v)
        lax.fori_loop(0, CH // LANES, pix_body, 0)
        pltpu.sync_copy(out_v, out_hbm.at[gchunk])
        return carry

    lax.fori_loop(0, CPW, chunk_body, 0)


_bilagrid_sc = functools.partial(
    pl.kernel,
    out_type=jax.ShapeDtypeStruct((N * CPV, 3, CH), jnp.float32),
    mesh=plsc.VectorSubcoreMesh(core_axis_name="c", subcore_axis_name="s"),
    compiler_params=pltpu.CompilerParams(needs_layout_passes=False),
    scratch_types=[
        pltpu.VMEM((NCH * NCELL,), jnp.float32),
        pltpu.VMEM((5, CH), jnp.float32),
        pltpu.VMEM((3, CH), jnp.float32),
    ],
)(_sc_body)


def kernel(grids, grid_xy, rgb):
    xy = grid_xy.reshape(N, CPV, CH, 2).transpose(0, 1, 3, 2)
    rgbt = rgb.reshape(N, CPV, CH, 3).transpose(0, 1, 3, 2)
    data = jnp.concatenate([xy, rgbt], axis=2).reshape(N * CPV, 5, CH)
    g = grids.reshape(N, NCH * NCELL)
    out = _bilagrid_sc(data, g)
    out = out.reshape(N, CPV, 3, CH).transpose(0, 1, 3, 2)
    return out.reshape(rgb.shape)


# speedup vs baseline: 11.9419x; 1.0129x over previous
import functools

import jax
import jax.numpy as jnp
from jax import lax
from jax.experimental import pallas as pl
from jax.experimental.pallas import tpu as pltpu
from jax.experimental.pallas import tpu_sc as plsc

N = 4
GL, GH, GW = 8, 16, 16
NCELL = GL * GH * GW
NCH = 12
P = 512 * 512
NWORKERS = 32
WPV = NWORKERS // N
CH = 4096
CPV = P // CH
CPW = CPV // WPV
LANES = 16


def _sc_body(data_hbm, grids_hbm, out_hbm, grid_v, in_v, out_v):
    cid = lax.axis_index("c")
    sid = lax.axis_index("s")
    wid = sid * 2 + cid
    view = wid // WPV
    slot = wid % WPV

    pltpu.sync_copy(grids_hbm.at[view], grid_v)

    coffs = [jnp.full((LANES,), c * NCELL, jnp.int32) for c in range(NCH)]

    def pix_body(i, carry):
        s = pl.ds(i * LANES, LANES)
        xv = in_v[0, s] * float(GW - 1)
        yv = in_v[1, s] * float(GH - 1)
        rv = in_v[2, s]
        gv = in_v[3, s]
        bv = in_v[4, s]
        gray = rv * 0.299 + gv * 0.587 + bv * 0.114
        zv = gray * float(GL - 1)

        x0 = xv.astype(jnp.int32)
        y0 = yv.astype(jnp.int32)
        z0 = zv.astype(jnp.int32)
        wx = xv - x0.astype(jnp.float32)
        wy = yv - y0.astype(jnp.float32)
        wz = zv - z0.astype(jnp.float32)
        x0c = jnp.minimum(x0, GW - 1)
        x1c = jnp.minimum(x0 + 1, GW - 1)
        y0c = jnp.minimum(y0, GH - 1)
        y1c = jnp.minimum(y0 + 1, GH - 1)
        z0c = jnp.minimum(z0, GL - 1)
        z1c = jnp.minimum(z0 + 1, GL - 1)

        r0 = z0c * (GH * GW)
        r1 = z1c * (GH * GW)
        c0 = y0c * GW
        c1 = y1c * GW
        zy00 = r0 + c0
        zy01 = r0 + c1
        zy10 = r1 + c0
        zy11 = r1 + c1
        idxs = [zy00 + x0c, zy00 + x1c,
                zy01 + x0c, zy01 + x1c,
                zy10 + x0c, zy10 + x1c,
                zy11 + x0c, zy11 + x1c]

        ux = 1.0 - wx
        uy = 1.0 - wy
        uz = 1.0 - wz
        wzy00 = uz * uy
        wzy01 = uz * wy
        wzy10 = wz * uy
        wzy11 = wz * wy
        ws = [wzy00 * ux, wzy00 * wx,
              wzy01 * ux, wzy01 * wx,
              wzy10 * ux, wzy10 * wx,
              wzy11 * ux, wzy11 * wx]

        mats = []
        for c in range(NCH):
            acc = ws[0] * plsc.load_gather(grid_v, [coffs[c] + idxs[0]])
            for k in range(1, 8):
                acc = acc + ws[k] * plsc.load_gather(grid_v, [coffs[c] + idxs[k]])
            mats.append(acc)

        out_v[0, s] = mats[0] * rv + mats[1] * gv + mats[2] * bv + mats[3]
        out_v[1, s] = mats[4] * rv + mats[5] * gv + mats[6] * bv + mats[7]
        out_v[2, s] = mats[8] * rv + mats[9] * gv + mats[10] * bv + mats[11]
        return carry

    def chunk_body(ci, carry):
        gchunk = view * CPV + slot * CPW + ci
        pltpu.sync_copy(data_hbm.at[gchunk], in_v)
        lax.fori_loop(0, CH // LANES, pix_body, 0)
        pltpu.sync_copy(out_v, out_hbm.at[gchunk])
        return carry

    lax.fori_loop(0, CPW, chunk_body, 0)


_bilagrid_sc = functools.partial(
    pl.kernel,
    out_type=jax.ShapeDtypeStruct((N * CPV, 3, CH), jnp.float32),
    mesh=plsc.VectorSubcoreMesh(core_axis_name="c", subcore_axis_name="s"),
    compiler_params=pltpu.CompilerParams(needs_layout_passes=False),
    scratch_types=[
        pltpu.VMEM((NCH * NCELL,), jnp.float32),
        pltpu.VMEM((5, CH), jnp.float32),
        pltpu.VMEM((3, CH), jnp.float32),
    ],
)(_sc_body)


def kernel(grids, grid_xy, rgb):
    xy = grid_xy.reshape(N, CPV, CH, 2).transpose(0, 1, 3, 2)
    rgbt = rgb.reshape(N, CPV, CH, 3).transpose(0, 1, 3, 2)
    data = jnp.concatenate([xy, rgbt], axis=2).reshape(N * CPV, 5, CH)
    g = grids.reshape(N, NCH * NCELL)
    out = _bilagrid_sc(data, g)
    out = out.reshape(N, CPV, 3, CH).transpose(0, 1, 3, 2)
    return out.reshape(rgb.shape)
